# baseline (speedup 1.0000x reference)
import functools

import jax
import jax.numpy as jnp
from jax import lax
from jax.experimental import pallas as pl
from jax.experimental.pallas import tpu as pltpu
from jax.experimental.pallas import tpu_sc as plsc

_B, _F = 4096, 26
_D = 64
_NC, _NS = 2, 16
_NW = _NC * _NS
_TOTAL = _B * _F
_PER_W = _TOTAL // _NW
_CHUNK = 128
_NCH = _PER_W // _CHUNK


def _gather_body(ids_hbm, table_hbm, out_hbm, idx_v, rows_v, sem0, sem1):
    wid = lax.axis_index("s") * _NC + lax.axis_index("c")
    base = wid * _PER_W
    pltpu.sync_copy(ids_hbm.at[wid], idx_v)
    sems = (sem0, sem1)

    pltpu.async_copy(table_hbm.at[idx_v.at[0]], rows_v.at[0], sems[0])

    for j in range(_NCH):
        buf = j % 2
        if j + 1 < _NCH:
            pltpu.async_copy(
                table_hbm.at[idx_v.at[j + 1]], rows_v.at[1 - buf], sems[1 - buf]
            )
        pltpu.make_async_copy(
            table_hbm.at[idx_v.at[j]], rows_v.at[buf], sems[buf]
        ).wait()
        pltpu.sync_copy(
            rows_v.at[buf], out_hbm.at[pl.ds(base + j * _CHUNK, _CHUNK)]
        )


def _build():
    mesh = plsc.VectorSubcoreMesh(core_axis_name="c", subcore_axis_name="s")
    return pl.kernel(
        _gather_body,
        mesh=mesh,
        out_type=jax.ShapeDtypeStruct((_TOTAL, _D), jnp.float32),
        scratch_types=[
            pltpu.VMEM((_NCH, _CHUNK), jnp.int32),
            pltpu.VMEM((2, _CHUNK, _D), jnp.float32),
            pltpu.SemaphoreType.DMA,
            pltpu.SemaphoreType.DMA,
        ],
        compiler_params=pltpu.CompilerParams(use_tc_tiling_on_sc=False),
    )


@jax.jit
def kernel(ids, table):
    ids3 = ids.reshape(_NW, _NCH, _CHUNK)
    out = _build()(ids3, table)
    return out.reshape(_B, _F, _D)


# speedup vs baseline: 1.0046x; 1.0046x over previous
import functools

import jax
import jax.numpy as jnp
from jax import lax
from jax.experimental import pallas as pl
from jax.experimental.pallas import tpu as pltpu
from jax.experimental.pallas import tpu_sc as plsc

_B, _F = 4096, 26
_D = 64
_NC, _NS = 2, 16
_NW = _NC * _NS
_TOTAL = _B * _F
_PER_W = _TOTAL // _NW
_CHUNK = 128
_NCH = _PER_W // _CHUNK


_NBUF = 13


def _gather_body(ids_hbm, table_hbm, out_hbm, idx_v, rows_v, *sems):
    gsem = sems[:_NBUF]
    osem = sems[_NBUF:]
    wid = lax.axis_index("s") * _NC + lax.axis_index("c")
    base = wid * _PER_W
    pltpu.sync_copy(ids_hbm.at[wid], idx_v)

    for j in range(_NBUF):
        pltpu.async_copy(table_hbm.at[idx_v.at[j]], rows_v.at[j], gsem[j])

    tail = []
    for j in range(_NCH):
        b = j % _NBUF
        pltpu.make_async_copy(
            table_hbm.at[idx_v.at[j]], rows_v.at[b], gsem[b]
        ).wait()
        out_slice = out_hbm.at[pl.ds(base + j * _CHUNK, _CHUNK)]
        pltpu.async_copy(rows_v.at[b], out_slice, osem[b])
        nj = j + _NBUF
        if nj < _NCH:
            pltpu.make_async_copy(rows_v.at[b], out_slice, osem[b]).wait()
            pltpu.async_copy(table_hbm.at[idx_v.at[nj]], rows_v.at[b], gsem[b])
        else:
            tail.append((rows_v.at[b], out_slice, osem[b]))

    for src, dst, sem in tail:
        pltpu.make_async_copy(src, dst, sem).wait()


def _build():
    mesh = plsc.VectorSubcoreMesh(core_axis_name="c", subcore_axis_name="s")
    return pl.kernel(
        _gather_body,
        mesh=mesh,
        out_type=jax.ShapeDtypeStruct((_TOTAL, _D), jnp.float32),
        scratch_types=[
            pltpu.VMEM((_NCH, _CHUNK), jnp.int32),
            pltpu.VMEM((_NBUF, _CHUNK, _D), jnp.float32),
        ] + [pltpu.SemaphoreType.DMA] * (2 * _NBUF),
        compiler_params=pltpu.CompilerParams(use_tc_tiling_on_sc=False),
    )


@jax.jit
def kernel(ids, table):
    ids3 = ids.reshape(_NW, _NCH, _CHUNK)
    out = _build()(ids3, table)
    return out.reshape(_B, _F, _D)
